# baseline (speedup 1.0000x reference)
import jax
import jax.numpy as jnp
from jax.experimental import pallas as pl
from jax.experimental.pallas import tpu as pltpu


def _stream_kernel(res_ref, x_ref, ptx_ref, xi_ref, ptx_out_ref):
    xi_ref[...] = x_ref[...] + res_ref[0]
    ptx_out_ref[...] = ptx_ref[...]


def kernel(x, ptx, bs, height, width, point_edges, point_src_dirs, point_tgt_dirs):
    C, H, W = x.shape[1], x.shape[2], x.shape[3]
    hw = H * W
    n_pts = ptx.shape[0]
    residual = (
        (jnp.asarray(bs) - 1) + (jnp.asarray(height) - H) + (jnp.asarray(width) - W)
    ).astype(x.dtype)
    res = residual.reshape(1)

    x2 = x.reshape(C, hw)

    G = 32
    xb = hw // G
    pb = n_pts // 4

    xi, ptx_out = pl.pallas_call(
        _stream_kernel,
        grid=(G,),
        in_specs=[
            pl.BlockSpec(memory_space=pltpu.SMEM),
            pl.BlockSpec((C, xb), lambda i: (0, i)),
            pl.BlockSpec((pb, C), lambda i: (i // 8, 0)),
        ],
        out_specs=[
            pl.BlockSpec((C, xb), lambda i: (0, i)),
            pl.BlockSpec((pb, C), lambda i: (i // 8, 0)),
        ],
        out_shape=[
            jax.ShapeDtypeStruct((C, hw), x.dtype),
            jax.ShapeDtypeStruct((n_pts, C), ptx.dtype),
        ],
    )(res, x2, ptx)

    return (xi.reshape(1, C, H, W), ptx_out)


# speedup vs baseline: 1.1159x; 1.1159x over previous
import jax
import jax.numpy as jnp
from jax.experimental import pallas as pl
from jax.experimental.pallas import tpu as pltpu


def _stream_kernel(res_ref, x_ref, ptx_ref, xi_ref, ptx_out_ref):
    xi_ref[...] = x_ref[...] + res_ref[0]
    ptx_out_ref[...] = ptx_ref[...]


def kernel(x, ptx, bs, height, width, point_edges, point_src_dirs, point_tgt_dirs):
    C, H, W = x.shape[1], x.shape[2], x.shape[3]
    hw = H * W
    n_pts = ptx.shape[0]
    residual = (
        (jnp.asarray(bs) - 1) + (jnp.asarray(height) - H) + (jnp.asarray(width) - W)
    ).astype(x.dtype)
    res = residual.reshape(1)

    x2 = x.reshape(C, hw)

    G = 16
    xb = C // G
    pb = n_pts // 4

    xi, ptx_out = pl.pallas_call(
        _stream_kernel,
        grid=(G,),
        in_specs=[
            pl.BlockSpec(memory_space=pltpu.SMEM),
            pl.BlockSpec((xb, hw), lambda i: (i, 0)),
            pl.BlockSpec((pb, C), lambda i: (i // 4, 0)),
        ],
        out_specs=[
            pl.BlockSpec((xb, hw), lambda i: (i, 0)),
            pl.BlockSpec((pb, C), lambda i: (i // 4, 0)),
        ],
        out_shape=[
            jax.ShapeDtypeStruct((C, hw), x.dtype),
            jax.ShapeDtypeStruct((n_pts, C), ptx.dtype),
        ],
    )(res, x2, ptx)

    return (xi.reshape(1, C, H, W), ptx_out)


# speedup vs baseline: 3.0780x; 2.7583x over previous
import jax
import jax.numpy as jnp
from jax.experimental import pallas as pl
from jax.experimental.pallas import tpu as pltpu


def _stream_kernel(res_ref, x_ref, ptx_ref, xi_ref, ptx_out_ref):
    xi_ref[...] = x_ref[...] + res_ref[0]
    ptx_out_ref[...] = ptx_ref[...]


def kernel(x, ptx, bs, height, width, point_edges, point_src_dirs, point_tgt_dirs):
    C, H, W = x.shape[1], x.shape[2], x.shape[3]
    n_pts = ptx.shape[0]
    residual = (
        (jnp.asarray(bs) - 1) + (jnp.asarray(height) - H) + (jnp.asarray(width) - W)
    ).astype(x.dtype)
    res = residual.reshape(1)

    G = 16
    xb = C // G
    pb = n_pts // 4

    xi, ptx_out = pl.pallas_call(
        _stream_kernel,
        grid=(G,),
        in_specs=[
            pl.BlockSpec(memory_space=pltpu.SMEM),
            pl.BlockSpec((1, xb, H, W), lambda i: (0, i, 0, 0)),
            pl.BlockSpec((pb, C), lambda i: (i // 4, 0)),
        ],
        out_specs=[
            pl.BlockSpec((1, xb, H, W), lambda i: (0, i, 0, 0)),
            pl.BlockSpec((pb, C), lambda i: (i // 4, 0)),
        ],
        out_shape=[
            jax.ShapeDtypeStruct((1, C, H, W), x.dtype),
            jax.ShapeDtypeStruct((n_pts, C), ptx.dtype),
        ],
    )(res, x, ptx)

    return (xi, ptx_out)
